# baseline (speedup 1.0000x reference)
import jax
import jax.numpy as jnp
from jax import lax
from jax.experimental import pallas as pl
from jax.experimental.pallas import tpu as pltpu
from jax.experimental.pallas import tpu_sc as plsc

NB = 4
NN = 10000
NE = 160000
ND = 128
NK = 4
NP = 10240
SL = NP // 16
EPT = NE // 16
ER = 79
EPP = ER * 128
FE = NB * NE // 32
FH = FE // 2


def _proj_body(sv_ref, w_ref, ch_ref, u_ref):
    avec = jnp.dot(sv_ref[...], w_ref[...], preferred_element_type=jnp.float32)
    u_ref[...] = lax.dot_general(
        avec, ch_ref[...], (((1,), (1,)), ((), ())),
        preferred_element_type=jnp.float32)


_proj = pl.pallas_call(
    _proj_body,
    out_shape=jax.ShapeDtypeStruct((2, NP), jnp.float32),
)


def _sigmoid_sum_into(dst, b):
    one16 = jnp.ones((16,), jnp.float32)

    def body(j, carry):
        for c in range(5):
            sl_ = pl.ds(j * 80 + c * 16, 16)
            x = dst[sl_] + b[sl_]
            dst[sl_] = one16 / (one16 + jnp.exp(-x))
        return carry

    lax.fori_loop(0, FH // 80, body, 0)


def _sc_body(u_hbm, hd_hbm, tl_hbm, tmp_hbm, out_hbm,
             he, te, he2, te2, m0, m1, mx,
             st0, st1, d2s, dvs, x0s, x1s, h0s, h1s, zbuf, tvm,
             s0, s1, s2, s3,
             zs0, zs1, a0, a1):
    cid = lax.axis_index("c")
    sid = lax.axis_index("s")
    base = sid * SL
    wid = cid * 16 + sid

    zeros16 = jnp.zeros((16,), jnp.float32)
    one16 = jnp.ones((16,), jnp.float32)
    half16 = jnp.full((16,), 0.5, jnp.float32)
    thalf16 = jnp.full((16,), 1.5, jnp.float32)

    pltpu.sync_copy(hd_hbm.at[pl.ds(sid * EPT, EPT)], he.at[pl.ds(0, EPT)])
    pltpu.sync_copy(tl_hbm.at[pl.ds(sid * EPT, EPT)], te.at[pl.ds(0, EPT)])
    pltpu.sync_copy(tmp_hbm, tvm)
    pltpu.sync_copy(hd_hbm.at[pl.ds(wid * FE, FH)], he2.at[pl.ds(0, FH)])
    pltpu.sync_copy(tl_hbm.at[pl.ds(wid * FE, FH)], te2.at[pl.ds(0, FH)])
    padn16 = jnp.full((16,), NP - 1, jnp.int32)
    zeroi16 = jnp.zeros((16,), jnp.int32)
    for i in range((EPP - EPT) // 16):
        he[pl.ds(EPT + i * 16, 16)] = padn16
        te[pl.ds(EPT + i * 16, 16)] = padn16
        he2[pl.ds(FH + i * 16, 16)] = zeroi16
        te2[pl.ds(FH + i * 16, 16)] = zeroi16
    for i in range(SL // 16):
        zbuf[pl.ds(i * 16, 16)] = zeros16

    pltpu.sync_copy(zbuf, a0.at[pl.ds(base, SL)])

    def _fill_ones(j, carry):
        for c in range(8):
            m0[pl.ds(j * 128 + c * 16, 16)] = one16
        return carry

    lax.fori_loop(0, ER, _fill_ones, 0)
    u0c = pltpu.async_copy(u_hbm.at[pl.ds(base, SL)], x0s, s0)
    u1c = pltpu.async_copy(u_hbm.at[pl.ds(NP + base, SL)], x1s, s1)
    plsc.subcore_barrier()
    degc = pltpu.async_copy(m0, a0.at[te], add=True, sem=s2)
    u0c.wait()
    u1c.wait()
    degc.wait()
    plsc.subcore_barrier()

    pltpu.sync_copy(a0.at[pl.ds(base, SL)], st0)
    t0v = tvm[0]
    for i in range(SL // 16):
        sl_ = pl.ds(i * 16, 16)
        deg = st0[sl_] + one16
        d2s[sl_] = one16 / deg
        y = jnp.full((16,), 0.70710678, jnp.float32)
        for lv in range(1, 10):
            thr = jnp.full((16,), float(4 ** lv), jnp.float32)
            y = jnp.where(deg >= thr, y * half16, y)
        hx = half16 * deg
        for _ in range(5):
            y = y * (thalf16 - hx * y * y)
        dvs[sl_] = y
        h0s[sl_] = t0v * x0s[sl_]
        h1s[sl_] = t0v * x1s[sl_]
        st0[sl_] = y * x0s[sl_]
        st1[sl_] = y * x1s[sl_]
    p0 = pltpu.async_copy(st0, zs0.at[pl.ds(base, SL)], s0)
    p1 = pltpu.async_copy(st1, zs1.at[pl.ds(base, SL)], s1)
    p2 = pltpu.async_copy(zbuf, a0.at[pl.ds(base, SL)], s2)
    p3 = pltpu.async_copy(zbuf, a1.at[pl.ds(base, SL)], s3)
    p0.wait()
    p1.wait()
    p2.wait()
    p3.wait()
    plsc.subcore_barrier()

    for k in range(1, NK + 1):
        g0 = pltpu.async_copy(zs0.at[he], m0, s0)
        g1 = pltpu.async_copy(zs1.at[he], m1, s1)
        g0.wait()
        c0 = pltpu.async_copy(m0, a0.at[te], add=True, sem=s2)
        g1.wait()
        c1 = pltpu.async_copy(m1, a1.at[te], add=True, sem=s3)
        c0.wait()
        c1.wait()
        plsc.subcore_barrier()
        tkv = tvm[k]
        f0 = pltpu.async_copy(a0.at[pl.ds(base, SL)], st0, s0)
        f1 = pltpu.async_copy(a1.at[pl.ds(base, SL)], st1, s1)
        f0.wait()
        f1.wait()
        for i in range(SL // 16):
            sl_ = pl.ds(i * 16, 16)
            dv = dvs[sl_]
            xn0 = dv * st0[sl_] + d2s[sl_] * x0s[sl_]
            xn1 = dv * st1[sl_] + d2s[sl_] * x1s[sl_]
            h0s[sl_] = h0s[sl_] + tkv * xn0
            h1s[sl_] = h1s[sl_] + tkv * xn1
            x0s[sl_] = xn0
            x1s[sl_] = xn1
            st0[sl_] = dv * xn0
            st1[sl_] = dv * xn1
        if k < NK:
            p0 = pltpu.async_copy(st0, zs0.at[pl.ds(base, SL)], s0)
            p1 = pltpu.async_copy(st1, zs1.at[pl.ds(base, SL)], s1)
            p2 = pltpu.async_copy(zbuf, a0.at[pl.ds(base, SL)], s2)
            p3 = pltpu.async_copy(zbuf, a1.at[pl.ds(base, SL)], s3)
            p2.wait()
            p3.wait()
        else:
            p0 = pltpu.async_copy(h0s, zs0.at[pl.ds(base, SL)], s0)
            p1 = pltpu.async_copy(h1s, zs1.at[pl.ds(base, SL)], s1)
        p0.wait()
        p1.wait()
        plsc.subcore_barrier()

    hl1 = pltpu.async_copy(hd_hbm.at[pl.ds(wid * FE + FH, FH)],
                           he.at[pl.ds(0, FH)], s0)
    tl1 = pltpu.async_copy(tl_hbm.at[pl.ds(wid * FE + FH, FH)],
                           te.at[pl.ds(0, FH)], s1)
    ga = pltpu.async_copy(zs0.at[he2], m0, s2)
    gb = pltpu.async_copy(zs1.at[te2], m1, s3)
    ga.wait()
    gb.wait()
    _sigmoid_sum_into(m0, m1)
    w0 = pltpu.async_copy(m0.at[pl.ds(0, FH)],
                          out_hbm.at[pl.ds(wid * FE, FH)], s3)
    hl1.wait()
    tl1.wait()
    ga = pltpu.async_copy(zs0.at[he], m1, s0)
    gb = pltpu.async_copy(zs1.at[te], mx, s1)
    ga.wait()
    gb.wait()
    _sigmoid_sum_into(m1, mx)
    w1 = pltpu.async_copy(m1.at[pl.ds(0, FH)],
                          out_hbm.at[pl.ds(wid * FE + FH, FH)], s2)
    w0.wait()
    w1.wait()


_sc_call = pl.kernel(
    _sc_body,
    out_type=jax.ShapeDtypeStruct((NB * NE,), jnp.float32),
    mesh=plsc.VectorSubcoreMesh(core_axis_name="c", subcore_axis_name="s"),
    scratch_types=[
        pltpu.VMEM((EPP,), jnp.int32),
        pltpu.VMEM((EPP,), jnp.int32),
        pltpu.VMEM((EPP,), jnp.int32),
        pltpu.VMEM((EPP,), jnp.int32),
        pltpu.VMEM((EPP,), jnp.float32),
        pltpu.VMEM((EPP,), jnp.float32),
        pltpu.VMEM((EPP,), jnp.float32),
        pltpu.VMEM((SL,), jnp.float32),
        pltpu.VMEM((SL,), jnp.float32),
        pltpu.VMEM((SL,), jnp.float32),
        pltpu.VMEM((SL,), jnp.float32),
        pltpu.VMEM((SL,), jnp.float32),
        pltpu.VMEM((SL,), jnp.float32),
        pltpu.VMEM((SL,), jnp.float32),
        pltpu.VMEM((SL,), jnp.float32),
        pltpu.VMEM((SL,), jnp.float32),
        pltpu.VMEM((16, 16), jnp.float32),
        pltpu.SemaphoreType.DMA,
        pltpu.SemaphoreType.DMA,
        pltpu.SemaphoreType.DMA,
        pltpu.SemaphoreType.DMA,
        pltpu.VMEM_SHARED((NP,), jnp.float32),
        pltpu.VMEM_SHARED((NP,), jnp.float32),
        pltpu.VMEM_SHARED((NP,), jnp.float32),
        pltpu.VMEM_SHARED((NP,), jnp.float32),
    ],
)


def kernel(concept_hidden, relation_hidden, head, tail, triple_label, temp,
           W, s_src, s_tgt):
    del relation_hidden, triple_label
    ch0p = jnp.pad(concept_hidden[0], ((0, NP - NN), (0, 0)))
    sv = jnp.concatenate([s_src[0, 0], s_tgt[0, 0]], axis=0)
    u = _proj(sv, W, ch0p).reshape(2 * NP)

    tpad = jnp.pad(temp, (0, 16 - (NK + 1)))
    tmat = jnp.tile(tpad[:, None], (1, 16))

    out = _sc_call(u, head.reshape(NB * NE), tail.reshape(NB * NE), tmat)
    return out.reshape(NB, NE)


# speedup vs baseline: 1.0179x; 1.0179x over previous
import jax
import jax.numpy as jnp
from jax import lax
from jax.experimental import pallas as pl
from jax.experimental.pallas import tpu as pltpu
from jax.experimental.pallas import tpu_sc as plsc

NB = 4
NN = 10000
NE = 160000
ND = 128
NK = 4
NP = 10240
SL = NP // 16
EPT = NE // 16
ER = 79
EPP = ER * 128
FE = NB * NE // 32
FH = FE // 2


def _proj_body(sv_ref, w_ref, ch_ref, u_ref):
    avec = jnp.dot(sv_ref[...], w_ref[...], preferred_element_type=jnp.float32)
    u_ref[...] = lax.dot_general(
        avec, ch_ref[...], (((1,), (1,)), ((), ())),
        preferred_element_type=jnp.float32)


_proj = pl.pallas_call(
    _proj_body,
    out_shape=jax.ShapeDtypeStruct((2, NP), jnp.float32),
)


def _sigmoid_sum_into(dst, b):
    one16 = jnp.ones((16,), jnp.float32)

    def body(j, carry):
        for c in range(5):
            sl_ = pl.ds(j * 80 + c * 16, 16)
            x = dst[sl_] + b[sl_]
            dst[sl_] = one16 / (one16 + jnp.exp(-x))
        return carry

    lax.fori_loop(0, FH // 80, body, 0)


def _sc_body(u_hbm, hd_hbm, tl_hbm, tmp_hbm, out_hbm,
             he, te, he2, te2, m0, m1, mx, my,
             st0, st1, d2s, dvs, x0s, x1s, h0s, h1s, zbuf, tvm,
             s0, s1, s2, s3,
             zs0, zs1, a0, a1):
    cid = lax.axis_index("c")
    sid = lax.axis_index("s")
    base = sid * SL
    wid = cid * 16 + sid

    zeros16 = jnp.zeros((16,), jnp.float32)
    one16 = jnp.ones((16,), jnp.float32)
    half16 = jnp.full((16,), 0.5, jnp.float32)
    thalf16 = jnp.full((16,), 1.5, jnp.float32)

    pltpu.sync_copy(hd_hbm.at[pl.ds(sid * EPT, EPT)], he.at[pl.ds(0, EPT)])
    pltpu.sync_copy(tl_hbm.at[pl.ds(sid * EPT, EPT)], te.at[pl.ds(0, EPT)])
    pltpu.sync_copy(tmp_hbm, tvm)
    pltpu.sync_copy(hd_hbm.at[pl.ds(wid * FE, FH)], he2.at[pl.ds(0, FH)])
    pltpu.sync_copy(tl_hbm.at[pl.ds(wid * FE, FH)], te2.at[pl.ds(0, FH)])
    padn16 = jnp.full((16,), NP - 1, jnp.int32)
    zeroi16 = jnp.zeros((16,), jnp.int32)
    for i in range((EPP - EPT) // 16):
        he[pl.ds(EPT + i * 16, 16)] = padn16
        te[pl.ds(EPT + i * 16, 16)] = padn16
        he2[pl.ds(FH + i * 16, 16)] = zeroi16
        te2[pl.ds(FH + i * 16, 16)] = zeroi16
    for i in range(SL // 16):
        zbuf[pl.ds(i * 16, 16)] = zeros16

    pltpu.sync_copy(zbuf, a0.at[pl.ds(base, SL)])

    def _fill_ones(j, carry):
        for c in range(8):
            m0[pl.ds(j * 128 + c * 16, 16)] = one16
        return carry

    lax.fori_loop(0, ER, _fill_ones, 0)
    u0c = pltpu.async_copy(u_hbm.at[pl.ds(base, SL)], x0s, s0)
    u1c = pltpu.async_copy(u_hbm.at[pl.ds(NP + base, SL)], x1s, s1)
    plsc.subcore_barrier()
    degc = pltpu.async_copy(m0, a0.at[te], add=True, sem=s2)
    u0c.wait()
    u1c.wait()
    degc.wait()
    plsc.subcore_barrier()

    pltpu.sync_copy(a0.at[pl.ds(base, SL)], st0)
    t0v = tvm[0]
    for i in range(SL // 16):
        sl_ = pl.ds(i * 16, 16)
        deg = st0[sl_] + one16
        d2s[sl_] = one16 / deg
        y = jnp.full((16,), 0.70710678, jnp.float32)
        for lv in range(1, 10):
            thr = jnp.full((16,), float(4 ** lv), jnp.float32)
            y = jnp.where(deg >= thr, y * half16, y)
        hx = half16 * deg
        for _ in range(5):
            y = y * (thalf16 - hx * y * y)
        dvs[sl_] = y
        h0s[sl_] = t0v * x0s[sl_]
        h1s[sl_] = t0v * x1s[sl_]
        st0[sl_] = y * x0s[sl_]
        st1[sl_] = y * x1s[sl_]
    p0 = pltpu.async_copy(st0, zs0.at[pl.ds(base, SL)], s0)
    p1 = pltpu.async_copy(st1, zs1.at[pl.ds(base, SL)], s1)
    p2 = pltpu.async_copy(zbuf, a0.at[pl.ds(base, SL)], s2)
    p3 = pltpu.async_copy(zbuf, a1.at[pl.ds(base, SL)], s3)
    p0.wait()
    p1.wait()
    p2.wait()
    p3.wait()
    plsc.subcore_barrier()

    for k in range(1, NK + 1):
        g0 = pltpu.async_copy(zs0.at[he], m0, s0)
        g1 = pltpu.async_copy(zs1.at[he], m1, s1)
        g0.wait()
        c0 = pltpu.async_copy(m0, a0.at[te], add=True, sem=s2)
        g1.wait()
        c1 = pltpu.async_copy(m1, a1.at[te], add=True, sem=s3)
        c0.wait()
        c1.wait()
        plsc.subcore_barrier()
        tkv = tvm[k]
        f0 = pltpu.async_copy(a0.at[pl.ds(base, SL)], st0, s0)
        f1 = pltpu.async_copy(a1.at[pl.ds(base, SL)], st1, s1)
        f0.wait()
        f1.wait()
        for i in range(SL // 16):
            sl_ = pl.ds(i * 16, 16)
            dv = dvs[sl_]
            xn0 = dv * st0[sl_] + d2s[sl_] * x0s[sl_]
            xn1 = dv * st1[sl_] + d2s[sl_] * x1s[sl_]
            h0s[sl_] = h0s[sl_] + tkv * xn0
            h1s[sl_] = h1s[sl_] + tkv * xn1
            x0s[sl_] = xn0
            x1s[sl_] = xn1
            st0[sl_] = dv * xn0
            st1[sl_] = dv * xn1
        if k < NK:
            p0 = pltpu.async_copy(st0, zs0.at[pl.ds(base, SL)], s0)
            p1 = pltpu.async_copy(st1, zs1.at[pl.ds(base, SL)], s1)
            p2 = pltpu.async_copy(zbuf, a0.at[pl.ds(base, SL)], s2)
            p3 = pltpu.async_copy(zbuf, a1.at[pl.ds(base, SL)], s3)
            p2.wait()
            p3.wait()
        else:
            p0 = pltpu.async_copy(h0s, zs0.at[pl.ds(base, SL)], s0)
            p1 = pltpu.async_copy(h1s, zs1.at[pl.ds(base, SL)], s1)
        p0.wait()
        p1.wait()
        plsc.subcore_barrier()

    hl1 = pltpu.async_copy(hd_hbm.at[pl.ds(wid * FE + FH, FH)],
                           he.at[pl.ds(0, FH)], s0)
    tl1 = pltpu.async_copy(tl_hbm.at[pl.ds(wid * FE + FH, FH)],
                           te.at[pl.ds(0, FH)], s1)
    ga = pltpu.async_copy(zs0.at[he2], m0, s2)
    gb = pltpu.async_copy(zs1.at[te2], m1, s3)
    ga.wait()
    gb.wait()
    hl1.wait()
    tl1.wait()
    gc = pltpu.async_copy(zs0.at[he], mx, s0)
    gd = pltpu.async_copy(zs1.at[te], my, s1)
    _sigmoid_sum_into(m0, m1)
    w0 = pltpu.async_copy(m0.at[pl.ds(0, FH)],
                          out_hbm.at[pl.ds(wid * FE, FH)], s3)
    gc.wait()
    gd.wait()
    _sigmoid_sum_into(mx, my)
    w1 = pltpu.async_copy(mx.at[pl.ds(0, FH)],
                          out_hbm.at[pl.ds(wid * FE + FH, FH)], s2)
    w0.wait()
    w1.wait()


_sc_call = pl.kernel(
    _sc_body,
    out_type=jax.ShapeDtypeStruct((NB * NE,), jnp.float32),
    mesh=plsc.VectorSubcoreMesh(core_axis_name="c", subcore_axis_name="s"),
    scratch_types=[
        pltpu.VMEM((EPP,), jnp.int32),
        pltpu.VMEM((EPP,), jnp.int32),
        pltpu.VMEM((EPP,), jnp.int32),
        pltpu.VMEM((EPP,), jnp.int32),
        pltpu.VMEM((EPP,), jnp.float32),
        pltpu.VMEM((EPP,), jnp.float32),
        pltpu.VMEM((EPP,), jnp.float32),
        pltpu.VMEM((EPP,), jnp.float32),
        pltpu.VMEM((SL,), jnp.float32),
        pltpu.VMEM((SL,), jnp.float32),
        pltpu.VMEM((SL,), jnp.float32),
        pltpu.VMEM((SL,), jnp.float32),
        pltpu.VMEM((SL,), jnp.float32),
        pltpu.VMEM((SL,), jnp.float32),
        pltpu.VMEM((SL,), jnp.float32),
        pltpu.VMEM((SL,), jnp.float32),
        pltpu.VMEM((SL,), jnp.float32),
        pltpu.VMEM((16, 16), jnp.float32),
        pltpu.SemaphoreType.DMA,
        pltpu.SemaphoreType.DMA,
        pltpu.SemaphoreType.DMA,
        pltpu.SemaphoreType.DMA,
        pltpu.VMEM_SHARED((NP,), jnp.float32),
        pltpu.VMEM_SHARED((NP,), jnp.float32),
        pltpu.VMEM_SHARED((NP,), jnp.float32),
        pltpu.VMEM_SHARED((NP,), jnp.float32),
    ],
)


def kernel(concept_hidden, relation_hidden, head, tail, triple_label, temp,
           W, s_src, s_tgt):
    del relation_hidden, triple_label
    ch0p = jnp.pad(concept_hidden[0], ((0, NP - NN), (0, 0)))
    sv = jnp.concatenate([s_src[0, 0], s_tgt[0, 0]], axis=0)
    u = _proj(sv, W, ch0p).reshape(2 * NP)

    tpad = jnp.pad(temp, (0, 16 - (NK + 1)))
    tmat = jnp.tile(tpad[:, None], (1, 16))

    out = _sc_call(u, head.reshape(NB * NE), tail.reshape(NB * NE), tmat)
    return out.reshape(NB, NE)
